# baseline (speedup 1.0000x reference)
import functools

import jax
import jax.numpy as jnp
from jax import lax
from jax.experimental import pallas as pl
from jax.experimental.pallas import tpu as pltpu
from jax.experimental.pallas import tpu_sc as plsc

_NQ = 1024
_NDB = 100000
_D = 128
_K = 5
_KPAD = 8
_GAMMA = 0.6
_BN = 1024

_NW = 32
_RPW = (_NQ * _K) // _NW
_RPC = _RPW // 2
_QPW = _NQ // _NW


def _norm_body(x_ref, o_ref):
    x = x_ref[...]
    n = jnp.sqrt(jnp.sum(x * x, axis=1, keepdims=True))
    o_ref[...] = x / jnp.maximum(n, 1e-12)


def _l2norm(x):
    return pl.pallas_call(
        _norm_body,
        out_shape=jax.ShapeDtypeStruct(x.shape, x.dtype),
    )(x)


def _extract_top5(smask, colf, j):
    nchunk = _BN // 128
    lane = colf[:, 0:128]
    vals, idxs = [], []
    work = smask
    for t in range(_K):
        gmax = work[:, 0:128]
        gcol = lane
        for c in range(1, nchunk):
            v = work[:, c * 128:(c + 1) * 128]
            better = v > gmax
            gcol = jnp.where(better, lane + jnp.float32(c * 128), gcol)
            gmax = jnp.where(better, v, gmax)
        m = jnp.max(gmax, axis=1, keepdims=True)
        fi = jnp.min(jnp.where(gmax == m, gcol, jnp.float32(_BN)),
                     axis=1, keepdims=True)
        vals.append(m)
        idxs.append(fi.astype(jnp.int32) + j * _BN)
        if t < _K - 1:
            work = jnp.where(colf == fi, -jnp.inf, work)
    return vals, idxs


def _score_topk_body(has_blend, qn_ref, db_ref, *rest):
    if has_blend:
        qn0_ref, s_ref, f_ref, vals_ref, idx_ref = rest
    else:
        s_ref, vals_ref, idx_ref = rest
    j = pl.program_id(0)

    db = db_ref[...]
    nrm = jnp.sqrt(jnp.sum(db * db, axis=1, keepdims=True))
    dbn = db / jnp.maximum(nrm, 1e-12)
    dbn16 = dbn.astype(jnp.bfloat16)
    s = lax.dot_general(
        qn_ref[...].astype(jnp.bfloat16), dbn16,
        (((1,), (1,)), ((), ())),
        preferred_element_type=jnp.float32,
    )
    s_ref[...] = s
    if has_blend:
        s_init = lax.dot_general(
            qn0_ref[...].astype(jnp.bfloat16), dbn16,
            (((1,), (1,)), ((), ())),
            preferred_element_type=jnp.float32,
        )
        f_ref[...] = _GAMMA * s_init + (1.0 - _GAMMA) * s

    colf = lax.broadcasted_iota(jnp.int32, (_NQ, _BN), 1).astype(jnp.float32)
    jbase = (j * _BN).astype(jnp.float32)
    smask = jnp.where(colf + jbase < jnp.float32(_NDB), s, -jnp.inf)
    tvals, tidxs = _extract_top5(smask, colf, j)
    pad_v = jnp.full((_NQ, _KPAD - _K), -jnp.inf, jnp.float32)
    pad_i = jnp.full((_NQ, _KPAD - _K), _NDB + 1, jnp.int32)
    tile_v = jnp.concatenate(tvals + [pad_v], axis=1)
    tile_i = jnp.concatenate(tidxs + [pad_i], axis=1)

    @pl.when(j == 0)
    def _init():
        vals_ref[...] = jnp.full((_NQ, _KPAD), -jnp.inf, jnp.float32)
        idx_ref[...] = jnp.full((_NQ, _KPAD), _NDB + 1, jnp.int32)

    cv = jnp.concatenate([vals_ref[...], tile_v], axis=1)
    ci = jnp.concatenate([idx_ref[...], tile_i], axis=1)
    nv, ni = [], []
    for t in range(_K):
        m = jnp.max(cv, axis=1, keepdims=True)
        fi = jnp.min(jnp.where(cv == m, ci, _NDB + 1), axis=1, keepdims=True)
        nv.append(m)
        ni.append(fi)
        cv = jnp.where(ci == fi, -jnp.inf, cv)
    vals_ref[...] = jnp.concatenate(nv + [pad_v], axis=1)
    idx_ref[...] = jnp.concatenate(ni + [pad_i], axis=1)


def _score_topk(qn, db, qn0=None):
    has_blend = qn0 is not None
    nsteps = pl.cdiv(_NDB, _BN)
    in_specs = [
        pl.BlockSpec((_NQ, _D), lambda j: (0, 0)),
        pl.BlockSpec((_BN, _D), lambda j: (j, 0)),
    ]
    operands = [qn, db]
    out_shapes = [jax.ShapeDtypeStruct((_NQ, _NDB), jnp.float32)]
    out_specs = [pl.BlockSpec((_NQ, _BN), lambda j: (0, j))]
    if has_blend:
        in_specs.append(pl.BlockSpec((_NQ, _D), lambda j: (0, 0)))
        operands.append(qn0)
        out_shapes.append(jax.ShapeDtypeStruct((_NQ, _NDB), jnp.float32))
        out_specs.append(pl.BlockSpec((_NQ, _BN), lambda j: (0, j)))
    out_shapes += [
        jax.ShapeDtypeStruct((_NQ, _KPAD), jnp.float32),
        jax.ShapeDtypeStruct((_NQ, _KPAD), jnp.int32),
    ]
    out_specs += [
        pl.BlockSpec((_NQ, _KPAD), lambda j: (0, 0)),
        pl.BlockSpec((_NQ, _KPAD), lambda j: (0, 0)),
    ]
    return pl.pallas_call(
        functools.partial(_score_topk_body, has_blend),
        grid=(nsteps,),
        in_specs=in_specs,
        out_specs=out_specs,
        out_shape=out_shapes,
        compiler_params=pltpu.CompilerParams(
            dimension_semantics=("arbitrary",),
        ),
    )(*operands)


def _sc_gather_mean(db, idx_flat):
    mesh = plsc.VectorSubcoreMesh(core_axis_name="c", subcore_axis_name="s")

    @functools.partial(
        pl.kernel,
        mesh=mesh,
        out_type=jax.ShapeDtypeStruct((_NQ, _D), jnp.float32),
        scratch_types=[
            pltpu.VMEM((_RPC,), jnp.int32),
            pltpu.VMEM((_RPC,), jnp.int32),
            pltpu.VMEM((_RPC, _D), jnp.float32),
            pltpu.VMEM((_RPC, _D), jnp.float32),
            pltpu.VMEM((_QPW, _D), jnp.float32),
            pltpu.SemaphoreType.DMA,
        ],
    )
    def gather_mean(db_hbm, idx_hbm, out_hbm, idx_a, idx_b, rows_a, rows_b,
                    out_v, sem):
        wid = lax.axis_index("s") * 2 + lax.axis_index("c")
        rbase = wid * _RPW
        pltpu.sync_copy(idx_hbm.at[pl.ds(rbase, _RPC)], idx_a)
        pltpu.sync_copy(idx_hbm.at[pl.ds(rbase + _RPC, _RPC)], idx_b)
        ca = pltpu.async_copy(db_hbm.at[idx_a], rows_a, sem)
        cb = pltpu.async_copy(db_hbm.at[idx_b], rows_b, sem)
        ca.wait()
        cb.wait()
        qpc = _QPW // 2

        def mean_chunk(rows_v, qoff, qi):
            for c in range(_D // 16):
                acc = rows_v[qi * _K, pl.ds(c * 16, 16)]
                for r in range(1, _K):
                    acc = acc + rows_v[qi * _K + r, pl.ds(c * 16, 16)]
                out_v[qoff + qi, pl.ds(c * 16, 16)] = acc / 5.0

        def body_a(qi, carry):
            mean_chunk(rows_a, 0, qi)
            return carry

        def body_b(qi, carry):
            mean_chunk(rows_b, qpc, qi)
            return carry

        lax.fori_loop(0, qpc, body_a, 0)
        lax.fori_loop(0, qpc, body_b, 0)
        pltpu.sync_copy(out_v, out_hbm.at[pl.ds(wid * _QPW, _QPW)])

    return gather_mean(db, idx_flat)


def _mlp_body(q_ref, a_ref, w1_ref, b1_ref, w2_ref, b2_ref, o_ref):
    dn = (((1,), (0,)), ((), ()))
    x = jnp.concatenate([q_ref[...], a_ref[...]], axis=1)
    h = lax.dot_general(x.astype(jnp.bfloat16),
                        w1_ref[...].astype(jnp.bfloat16), dn,
                        preferred_element_type=jnp.float32)
    h = jnp.maximum(h + b1_ref[...], 0.0)
    o = lax.dot_general(h.astype(jnp.bfloat16),
                        w2_ref[...].astype(jnp.bfloat16), dn,
                        preferred_element_type=jnp.float32)
    o_ref[...] = o + b2_ref[...]


def _mlp(q, avg, W1, b1, W2, b2):
    return pl.pallas_call(
        _mlp_body,
        out_shape=jax.ShapeDtypeStruct((_NQ, _D), jnp.float32),
    )(q, avg, W1, b1.reshape(1, _D), W2, b2.reshape(1, _D))


def kernel(query_features, database_features, W1, b1, W2, b2):
    db = database_features
    qn = _l2norm(query_features)
    s0, v0, i0 = _score_topk(qn, db)
    avg0 = _sc_gather_mean(db, i0[:, :_K].reshape(_NQ * _K))
    eq0 = _mlp(query_features, avg0, W1, b1, W2, b2)
    en = _l2norm(eq0)
    e0, f0, v1, i1 = _score_topk(en, db, qn)
    avg1 = _sc_gather_mean(db, i1[:, :_K].reshape(_NQ * _K))
    eq1 = _mlp(eq0, avg1, W1, b1, W2, b2)
    return eq1, s0, e0, f0


# speedup vs baseline: 1.1206x; 1.1206x over previous
import functools

import jax
import jax.numpy as jnp
from jax import lax
from jax.experimental import pallas as pl
from jax.experimental.pallas import tpu as pltpu
from jax.experimental.pallas import tpu_sc as plsc

_NQ = 1024
_NDB = 100000
_D = 128
_K = 5
_KPAD = 8
_GAMMA = 0.6
_BN = 2048

_NW = 32
_RPW = (_NQ * _K) // _NW
_RPC = _RPW // 2
_QPW = _NQ // _NW


def _norm_body(x_ref, o_ref):
    x = x_ref[...]
    n = jnp.sqrt(jnp.sum(x * x, axis=1, keepdims=True))
    o_ref[...] = x / jnp.maximum(n, 1e-12)


def _l2norm(x):
    return pl.pallas_call(
        _norm_body,
        out_shape=jax.ShapeDtypeStruct(x.shape, x.dtype),
    )(x)


def _extract_top5(smask, colf, j):
    nchunk = _BN // 128
    lane = colf[:, 0:128]
    vals, idxs = [], []
    work = smask
    for t in range(_K):
        gmax = work[:, 0:128]
        gcol = lane
        for c in range(1, nchunk):
            v = work[:, c * 128:(c + 1) * 128]
            better = v > gmax
            gcol = jnp.where(better, lane + jnp.float32(c * 128), gcol)
            gmax = jnp.where(better, v, gmax)
        m = jnp.max(gmax, axis=1, keepdims=True)
        fi = jnp.min(jnp.where(gmax == m, gcol, jnp.float32(_BN)),
                     axis=1, keepdims=True)
        vals.append(m)
        idxs.append(fi.astype(jnp.int32) + j * _BN)
        if t < _K - 1:
            work = jnp.where(colf == fi, -jnp.inf, work)
    return vals, idxs


def _score_topk_body(has_blend, qn_ref, db_ref, *rest):
    if has_blend:
        qn0_ref, s_ref, f_ref, vals_ref, idx_ref = rest
    else:
        s_ref, vals_ref, idx_ref = rest
    j = pl.program_id(0)

    db = db_ref[...]
    nrm = jnp.sqrt(jnp.sum(db * db, axis=1, keepdims=True))
    dbn = db / jnp.maximum(nrm, 1e-12)
    dbn16 = dbn.astype(jnp.bfloat16)
    s = lax.dot_general(
        qn_ref[...].astype(jnp.bfloat16), dbn16,
        (((1,), (1,)), ((), ())),
        preferred_element_type=jnp.float32,
    )
    s_ref[...] = s
    if has_blend:
        s_init = lax.dot_general(
            qn0_ref[...].astype(jnp.bfloat16), dbn16,
            (((1,), (1,)), ((), ())),
            preferred_element_type=jnp.float32,
        )
        f_ref[...] = _GAMMA * s_init + (1.0 - _GAMMA) * s

    colf = lax.broadcasted_iota(jnp.int32, (_NQ, _BN), 1).astype(jnp.float32)
    jbase = (j * _BN).astype(jnp.float32)
    smask = jnp.where(colf + jbase < jnp.float32(_NDB), s, -jnp.inf)
    tvals, tidxs = _extract_top5(smask, colf, j)
    pad_v = jnp.full((_NQ, _KPAD - _K), -jnp.inf, jnp.float32)
    pad_i = jnp.full((_NQ, _KPAD - _K), _NDB + 1, jnp.int32)
    tile_v = jnp.concatenate(tvals + [pad_v], axis=1)
    tile_i = jnp.concatenate(tidxs + [pad_i], axis=1)

    @pl.when(j == 0)
    def _init():
        vals_ref[...] = jnp.full((_NQ, _KPAD), -jnp.inf, jnp.float32)
        idx_ref[...] = jnp.full((_NQ, _KPAD), _NDB + 1, jnp.int32)

    cv = jnp.concatenate([vals_ref[...], tile_v], axis=1)
    ci = jnp.concatenate([idx_ref[...], tile_i], axis=1)
    nv, ni = [], []
    for t in range(_K):
        m = jnp.max(cv, axis=1, keepdims=True)
        fi = jnp.min(jnp.where(cv == m, ci, _NDB + 1), axis=1, keepdims=True)
        nv.append(m)
        ni.append(fi)
        cv = jnp.where(ci == fi, -jnp.inf, cv)
    vals_ref[...] = jnp.concatenate(nv + [pad_v], axis=1)
    idx_ref[...] = jnp.concatenate(ni + [pad_i], axis=1)


def _score_topk(qn, db, qn0=None):
    has_blend = qn0 is not None
    nsteps = pl.cdiv(_NDB, _BN)
    in_specs = [
        pl.BlockSpec((_NQ, _D), lambda j: (0, 0)),
        pl.BlockSpec((_BN, _D), lambda j: (j, 0)),
    ]
    operands = [qn, db]
    out_shapes = [jax.ShapeDtypeStruct((_NQ, _NDB), jnp.float32)]
    out_specs = [pl.BlockSpec((_NQ, _BN), lambda j: (0, j))]
    if has_blend:
        in_specs.append(pl.BlockSpec((_NQ, _D), lambda j: (0, 0)))
        operands.append(qn0)
        out_shapes.append(jax.ShapeDtypeStruct((_NQ, _NDB), jnp.float32))
        out_specs.append(pl.BlockSpec((_NQ, _BN), lambda j: (0, j)))
    out_shapes += [
        jax.ShapeDtypeStruct((_NQ, _KPAD), jnp.float32),
        jax.ShapeDtypeStruct((_NQ, _KPAD), jnp.int32),
    ]
    out_specs += [
        pl.BlockSpec((_NQ, _KPAD), lambda j: (0, 0)),
        pl.BlockSpec((_NQ, _KPAD), lambda j: (0, 0)),
    ]
    return pl.pallas_call(
        functools.partial(_score_topk_body, has_blend),
        grid=(nsteps,),
        in_specs=in_specs,
        out_specs=out_specs,
        out_shape=out_shapes,
        compiler_params=pltpu.CompilerParams(
            dimension_semantics=("arbitrary",),
        ),
    )(*operands)


def _sc_gather_mean(db, idx_flat):
    mesh = plsc.VectorSubcoreMesh(core_axis_name="c", subcore_axis_name="s")

    @functools.partial(
        pl.kernel,
        mesh=mesh,
        out_type=jax.ShapeDtypeStruct((_NQ, _D), jnp.float32),
        scratch_types=[
            pltpu.VMEM((_RPC,), jnp.int32),
            pltpu.VMEM((_RPC,), jnp.int32),
            pltpu.VMEM((_RPC, _D), jnp.float32),
            pltpu.VMEM((_RPC, _D), jnp.float32),
            pltpu.VMEM((_QPW, _D), jnp.float32),
            pltpu.SemaphoreType.DMA,
        ],
    )
    def gather_mean(db_hbm, idx_hbm, out_hbm, idx_a, idx_b, rows_a, rows_b,
                    out_v, sem):
        wid = lax.axis_index("s") * 2 + lax.axis_index("c")
        rbase = wid * _RPW
        pltpu.sync_copy(idx_hbm.at[pl.ds(rbase, _RPC)], idx_a)
        pltpu.sync_copy(idx_hbm.at[pl.ds(rbase + _RPC, _RPC)], idx_b)
        ca = pltpu.async_copy(db_hbm.at[idx_a], rows_a, sem)
        cb = pltpu.async_copy(db_hbm.at[idx_b], rows_b, sem)
        ca.wait()
        cb.wait()
        qpc = _QPW // 2

        def mean_chunk(rows_v, qoff, qi):
            for c in range(_D // 16):
                acc = rows_v[qi * _K, pl.ds(c * 16, 16)]
                for r in range(1, _K):
                    acc = acc + rows_v[qi * _K + r, pl.ds(c * 16, 16)]
                out_v[qoff + qi, pl.ds(c * 16, 16)] = acc / 5.0

        def body_a(qi, carry):
            mean_chunk(rows_a, 0, qi)
            return carry

        def body_b(qi, carry):
            mean_chunk(rows_b, qpc, qi)
            return carry

        lax.fori_loop(0, qpc, body_a, 0)
        lax.fori_loop(0, qpc, body_b, 0)
        pltpu.sync_copy(out_v, out_hbm.at[pl.ds(wid * _QPW, _QPW)])

    return gather_mean(db, idx_flat)


def _mlp_body(q_ref, a_ref, w1_ref, b1_ref, w2_ref, b2_ref, o_ref):
    dn = (((1,), (0,)), ((), ()))
    x = jnp.concatenate([q_ref[...], a_ref[...]], axis=1)
    h = lax.dot_general(x.astype(jnp.bfloat16),
                        w1_ref[...].astype(jnp.bfloat16), dn,
                        preferred_element_type=jnp.float32)
    h = jnp.maximum(h + b1_ref[...], 0.0)
    o = lax.dot_general(h.astype(jnp.bfloat16),
                        w2_ref[...].astype(jnp.bfloat16), dn,
                        preferred_element_type=jnp.float32)
    o_ref[...] = o + b2_ref[...]


def _mlp(q, avg, W1, b1, W2, b2):
    return pl.pallas_call(
        _mlp_body,
        out_shape=jax.ShapeDtypeStruct((_NQ, _D), jnp.float32),
    )(q, avg, W1, b1.reshape(1, _D), W2, b2.reshape(1, _D))


def kernel(query_features, database_features, W1, b1, W2, b2):
    db = database_features
    qn = _l2norm(query_features)
    s0, v0, i0 = _score_topk(qn, db)
    avg0 = _sc_gather_mean(db, i0[:, :_K].reshape(_NQ * _K))
    eq0 = _mlp(query_features, avg0, W1, b1, W2, b2)
    en = _l2norm(eq0)
    e0, f0, v1, i1 = _score_topk(en, db, qn)
    avg1 = _sc_gather_mean(db, i1[:, :_K].reshape(_NQ * _K))
    eq1 = _mlp(eq0, avg1, W1, b1, W2, b2)
    return eq1, s0, e0, f0
